# baseline (speedup 1.0000x reference)
import functools

import jax
import jax.numpy as jnp
from jax import lax
from jax.experimental import pallas as pl
from jax.experimental.pallas import tpu as pltpu
from jax.experimental.pallas import tpu_sc as plsc

D = 32
NC = 2
NS = 16
NW = NC * NS
G = 50
L = 16


def _make_gather(NI):
    ipw = NI // NW
    mesh = plsc.VectorSubcoreMesh(core_axis_name="c", subcore_axis_name="s")

    @functools.partial(
        pl.kernel,
        mesh=mesh,
        out_type=jax.ShapeDtypeStruct((G, D, NI), jnp.float32),
        compiler_params=pltpu.CompilerParams(
            use_tc_tiling_on_sc=False, needs_layout_passes=False),
        scratch_types=[
            pltpu.VMEM((ipw,), jnp.int32),
            pltpu.VMEM((ipw,), jnp.int32),
            pltpu.VMEM((ipw, D), jnp.float32),
            pltpu.VMEM((ipw, D), jnp.float32),
            pltpu.VMEM((D, ipw), jnp.float32),
            pltpu.VMEM((D, ipw), jnp.float32),
            pltpu.SemaphoreType.DMA,
            pltpu.SemaphoreType.DMA,
            pltpu.SemaphoreType.DMA,
            pltpu.SemaphoreType.DMA,
            pltpu.SemaphoreType.DMA,
            pltpu.SemaphoreType.DMA,
        ],
    )
    def k(table, zt, outh, i0, i1, r0, r1, t0, t1,
          is0, is1, gs0, gs1, ss0, ss1):
        wid = lax.axis_index("s") * NC + lax.axis_index("c")
        col0 = wid * ipw
        idx = (i0, i1)
        rows = (r0, r1)
        rt = (t0, t1)
        isem = (is0, is1)
        gsem = (gs0, gs1)
        ssem = (ss0, ss1)

        def iload(j, b):
            return pltpu.make_async_copy(
                zt.at[j, pl.ds(col0, ipw)], idx[b], isem[b])

        def gath(b):
            return pltpu.make_async_copy(
                table.at[idx[b]], rows[b], gsem[b])

        def stor(j, b):
            return pltpu.make_async_copy(
                rt[b], outh.at[j, :, pl.ds(col0, ipw)], ssem[b])

        iota = lax.iota(jnp.int32, L)

        def transpose(b):
            src = rows[b]
            dst = rt[b]

            def tbody(c, carry):
                col = jnp.full((L,), c, jnp.int32)
                for blk in range(ipw // L):
                    v = plsc.load_gather(src, [blk * L + iota, col])
                    dst[c, pl.ds(blk * L, L)] = v
                return carry

            lax.fori_loop(0, D, tbody, 0)

        iload(0, 0).start()
        iload(1, 1).start()
        iload(0, 0).wait()
        gath(0).start()
        iload(1, 1).wait()
        gath(1).start()

        def step(j, b, first):
            gath(b).wait()
            if j + 2 < G:
                iload(j + 2, b).start()
            if not first:
                stor(j - 2, b).wait()
            transpose(b)
            stor(j, b).start()
            if j + 2 < G:
                iload(j + 2, b).wait()
                gath(b).start()

        step(0, 0, True)
        step(1, 1, True)

        def body(t, carry):
            j0 = 2 * t
            gath(0).wait()
            iload(j0 + 2, 0).start()
            stor(j0 - 2, 0).wait()
            transpose(0)
            stor(j0, 0).start()
            iload(j0 + 2, 0).wait()
            gath(0).start()

            j1 = j0 + 1
            gath(1).wait()
            iload(j1 + 2, 1).start()
            stor(j1 - 2, 1).wait()
            transpose(1)
            stor(j1, 1).start()
            iload(j1 + 2, 1).wait()
            gath(1).start()
            return carry

        lax.fori_loop(1, G // 2 - 1, body, 0)

        step(G - 2, 0, False)
        step(G - 1, 1, False)
        stor(G - 2, 0).wait()
        stor(G - 1, 1).wait()

    return k


def kernel(Z, embeddings):
    NI = Z.shape[0]
    zt = jnp.transpose(Z)
    out_phys = _make_gather(NI)(embeddings, zt)
    return jnp.transpose(out_phys, (2, 0, 1))


# speedup vs baseline: 1.6840x; 1.6840x over previous
import functools

import jax
import jax.numpy as jnp
from jax import lax
from jax.experimental import pallas as pl
from jax.experimental.pallas import tpu as pltpu
from jax.experimental.pallas import tpu_sc as plsc

D = 32
NC = 2
NS = 16
NW = NC * NS
G = 50
L = 16


def _make_gather(NI):
    ipw = NI // NW
    mesh = plsc.VectorSubcoreMesh(core_axis_name="c", subcore_axis_name="s")

    @functools.partial(
        pl.kernel,
        mesh=mesh,
        out_type=jax.ShapeDtypeStruct((G, D, NI), jnp.float32),
        compiler_params=pltpu.CompilerParams(
            use_tc_tiling_on_sc=False, needs_layout_passes=False),
        scratch_types=[
            pltpu.VMEM((ipw,), jnp.int32),
            pltpu.VMEM((ipw,), jnp.int32),
            pltpu.VMEM((ipw, D), jnp.float32),
            pltpu.VMEM((ipw, D), jnp.float32),
            pltpu.VMEM((D, ipw + 1), jnp.float32),
            pltpu.VMEM((D, ipw + 1), jnp.float32),
            pltpu.SemaphoreType.DMA,
            pltpu.SemaphoreType.DMA,
            pltpu.SemaphoreType.DMA,
            pltpu.SemaphoreType.DMA,
            pltpu.SemaphoreType.DMA,
            pltpu.SemaphoreType.DMA,
        ],
    )
    def k(table, zt, outh, i0, i1, r0, r1, t0, t1,
          is0, is1, gs0, gs1, ss0, ss1):
        wid = lax.axis_index("s") * NC + lax.axis_index("c")
        col0 = wid * ipw
        idx = (i0, i1)
        rows = (r0, r1)
        rt = (t0, t1)
        isem = (is0, is1)
        gsem = (gs0, gs1)
        ssem = (ss0, ss1)

        def iload(j, b):
            return pltpu.make_async_copy(
                zt.at[j, pl.ds(col0, ipw)], idx[b], isem[b])

        def gath(b):
            return pltpu.make_async_copy(
                table.at[idx[b]], rows[b], gsem[b])

        def stor(j, b):
            return pltpu.make_async_copy(
                rt[b].at[:, pl.ds(0, ipw)],
                outh.at[j, :, pl.ds(col0, ipw)], ssem[b])

        iota = lax.iota(jnp.int32, L)
        c_lo = iota
        c_hi = iota + L
        UNROLL = 8

        def transpose(b):
            src = rows[b]
            dst = rt[b]

            def tbody(r8, carry):
                r0 = r8 * UNROLL
                for u in range(UNROLL):
                    r = r0 + u
                    rvec = jnp.full((L,), r, jnp.int32)
                    v0 = src[r, pl.ds(0, L)]
                    v1 = src[r, pl.ds(L, L)]
                    plsc.store_scatter(dst, [c_lo, rvec], v0)
                    plsc.store_scatter(dst, [c_hi, rvec], v1)
                return carry

            lax.fori_loop(0, ipw // UNROLL, tbody, 0)

        iload(0, 0).start()
        iload(1, 1).start()
        iload(0, 0).wait()
        gath(0).start()
        iload(1, 1).wait()
        gath(1).start()

        def step(j, b, first):
            gath(b).wait()
            if j + 2 < G:
                iload(j + 2, b).start()
            if not first:
                stor(j - 2, b).wait()
            transpose(b)
            stor(j, b).start()
            if j + 2 < G:
                iload(j + 2, b).wait()
                gath(b).start()

        step(0, 0, True)
        step(1, 1, True)

        def body(t, carry):
            j0 = 2 * t
            gath(0).wait()
            iload(j0 + 2, 0).start()
            stor(j0 - 2, 0).wait()
            transpose(0)
            stor(j0, 0).start()
            iload(j0 + 2, 0).wait()
            gath(0).start()

            j1 = j0 + 1
            gath(1).wait()
            iload(j1 + 2, 1).start()
            stor(j1 - 2, 1).wait()
            transpose(1)
            stor(j1, 1).start()
            iload(j1 + 2, 1).wait()
            gath(1).start()
            return carry

        lax.fori_loop(1, G // 2 - 1, body, 0)

        step(G - 2, 0, False)
        step(G - 1, 1, False)
        stor(G - 2, 0).wait()
        stor(G - 1, 1).wait()

    return k


def kernel(Z, embeddings):
    NI = Z.shape[0]
    zt = jnp.transpose(Z)
    out_phys = _make_gather(NI)(embeddings, zt)
    return jnp.transpose(out_phys, (2, 0, 1))


# speedup vs baseline: 1.9348x; 1.1489x over previous
import functools

import jax
import jax.numpy as jnp
from jax import lax
from jax.experimental import pallas as pl
from jax.experimental.pallas import tpu as pltpu
from jax.experimental.pallas import tpu_sc as plsc

D = 32
NC = 2
NS = 16
NW = NC * NS
G = 50
L = 16


def _make_gather(NI):
    ipw = NI // NW
    mesh = plsc.VectorSubcoreMesh(core_axis_name="c", subcore_axis_name="s")

    @functools.partial(
        pl.kernel,
        mesh=mesh,
        out_type=jax.ShapeDtypeStruct((G, D // 8, NI // 128, 8, 128),
                                      jnp.float32),
        compiler_params=pltpu.CompilerParams(
            use_tc_tiling_on_sc=False, needs_layout_passes=False),
        scratch_types=[
            pltpu.VMEM((ipw,), jnp.int32),
            pltpu.VMEM((ipw,), jnp.int32),
            pltpu.VMEM((ipw, D), jnp.float32),
            pltpu.VMEM((ipw, D), jnp.float32),
            pltpu.VMEM((D, ipw + 1), jnp.float32),
            pltpu.VMEM((D, ipw + 1), jnp.float32),
            pltpu.SemaphoreType.DMA,
            pltpu.SemaphoreType.DMA,
            pltpu.SemaphoreType.DMA,
            pltpu.SemaphoreType.DMA,
            pltpu.SemaphoreType.DMA,
            pltpu.SemaphoreType.DMA,
        ],
    )
    def k(table, zt, outh, i0, i1, r0, r1, t0, t1,
          is0, is1, gs0, gs1, ss0, ss1):
        wid = lax.axis_index("s") * NC + lax.axis_index("c")
        col0 = wid * ipw
        idx = (i0, i1)
        rows = (r0, r1)
        rt = (t0, t1)
        isem = (is0, is1)
        gsem = (gs0, gs1)
        ssem = (ss0, ss1)

        def iload(j, b):
            return pltpu.make_async_copy(
                zt.at[j, pl.ds(col0, ipw)], idx[b], isem[b])

        def gath(b):
            return pltpu.make_async_copy(
                table.at[idx[b]], rows[b], gsem[b])

        tc0 = wid * (ipw // 128)

        def _stor_descs(j, b):
            for tr in range(D // 8):
                for tc in range(ipw // 128):
                    yield pltpu.make_async_copy(
                        rt[b].at[pl.ds(tr * 8, 8), pl.ds(tc * 128, 128)],
                        outh.at[j, tr, tc0 + tc], ssem[b])

        class _Stor:
            def __init__(self, j, b):
                self.j, self.b = j, b

            def start(self):
                for dsc in _stor_descs(self.j, self.b):
                    dsc.start()

            def wait(self):
                for dsc in _stor_descs(self.j, self.b):
                    dsc.wait()

        def stor(j, b):
            return _Stor(j, b)

        iota = lax.iota(jnp.int32, L)
        c_lo = iota
        c_hi = iota + L
        UNROLL = 8

        def transpose(b):
            src = rows[b]
            dst = rt[b]

            def tbody(r8, carry):
                r0 = r8 * UNROLL
                for u in range(UNROLL):
                    r = r0 + u
                    rvec = jnp.full((L,), r, jnp.int32)
                    v0 = src[r, pl.ds(0, L)]
                    v1 = src[r, pl.ds(L, L)]
                    plsc.store_scatter(dst, [c_lo, rvec], v0)
                    plsc.store_scatter(dst, [c_hi, rvec], v1)
                return carry

            lax.fori_loop(0, ipw // UNROLL, tbody, 0)

        iload(0, 0).start()
        iload(1, 1).start()
        iload(0, 0).wait()
        gath(0).start()
        iload(1, 1).wait()
        gath(1).start()

        def step(j, b, first):
            gath(b).wait()
            if j + 2 < G:
                iload(j + 2, b).start()
            if not first:
                stor(j - 2, b).wait()
            transpose(b)
            stor(j, b).start()
            if j + 2 < G:
                iload(j + 2, b).wait()
                gath(b).start()

        step(0, 0, True)
        step(1, 1, True)

        def body(t, carry):
            j0 = 2 * t
            gath(0).wait()
            iload(j0 + 2, 0).start()
            stor(j0 - 2, 0).wait()
            transpose(0)
            stor(j0, 0).start()
            iload(j0 + 2, 0).wait()
            gath(0).start()

            j1 = j0 + 1
            gath(1).wait()
            iload(j1 + 2, 1).start()
            stor(j1 - 2, 1).wait()
            transpose(1)
            stor(j1, 1).start()
            iload(j1 + 2, 1).wait()
            gath(1).start()
            return carry

        lax.fori_loop(1, G // 2 - 1, body, 0)

        step(G - 2, 0, False)
        step(G - 1, 1, False)
        stor(G - 2, 0).wait()
        stor(G - 1, 1).wait()

    return k


def kernel(Z, embeddings):
    NI = Z.shape[0]
    zt = jnp.transpose(Z)
    o5 = _make_gather(NI)(embeddings, zt)
    out_phys = jnp.transpose(o5, (0, 1, 3, 2, 4)).reshape(G, D, NI)
    return jnp.transpose(out_phys, (2, 0, 1))
